# baseline (speedup 1.0000x reference)
import functools

import jax
import jax.numpy as jnp
from jax import lax
from jax.experimental import pallas as pl
from jax.experimental.pallas import tpu as pltpu
from jax.experimental.pallas import tpu_sc as plsc

N = 25000
NP = 25088
E = 400000
EP = 425984
K = 128
CPT = 104
RPT = NP // 16
GPT = CPT // 8
TRASH = N
RB = 512
GRID = NP // RB
H = 64


def _dot(a, b):
    return jnp.dot(a, b, preferred_element_type=jnp.float32)


def _pre_body(me, ms, ma, dsm, das,
              Wue, bue, Wums, bums, Wuma, buma, Wuds, buds, Wuda, buda,
              Wle, ble, Wlms, blms, Wlma, blma, Wlds, blds, Wlda, blda,
              Wleu, bleu, Wr, br, scal,
              xomr_o, xodr_o, xum0_o, xum1_o, xud0_o, xud1_o):
    s = scal[...]
    me_, ms_, ma_, dsm_, das_ = me[...], ms[...], ma[...], dsm[...], das[...]
    emb_p = _dot(me_, Wle[...]) + ble[...]
    sim_p = _dot(ms_, Wlms[...]) + blms[...]
    ass_p = _dot(ma_, Wlma[...]) + blma[...]
    dsim_p = _dot(dsm_, Wlds[...]) + blds[...]
    dass_p = _dot(das_, Wlda[...]) + blda[...]
    xom = emb_p * s[0, 0] + sim_p * s[0, 1] + ass_p * s[0, 2]
    xod = dsim_p * s[0, 3] + dass_p * s[0, 4]
    xomr_o[...] = _dot(xom, Wr[...]) + br[...]
    xodr_o[...] = _dot(xod, Wr[...]) + br[...]
    ue = jnp.tanh(_dot(me_, Wue[...]) + bue[...])
    ums = jnp.tanh(_dot(ms_, Wums[...]) + bums[...])
    uma = jnp.tanh(_dot(ma_, Wuma[...]) + buma[...])
    uds = jnp.tanh(_dot(dsm_, Wuds[...]) + buds[...])
    uda = jnp.tanh(_dot(das_, Wuda[...]) + buda[...])
    emb_u = _dot(ue, Wleu[...]) + bleu[...]
    sim_u = _dot(ums, Wlms[...]) + blms[...]
    ass_u = _dot(uma, Wlma[...]) + blma[...]
    dsim_u = _dot(uds, Wlds[...]) + blds[...]
    dass_u = _dot(uda, Wlda[...]) + blda[...]
    xum = emb_u * s[0, 5] + sim_u * s[0, 6] + ass_u * s[0, 7]
    xud = dsim_u * s[0, 8] + dass_u * s[0, 9]
    xum0_o[...] = xum[:, :H]
    xum1_o[...] = xum[:, H:]
    xud0_o[...] = xud[:, :H]
    xud1_o[...] = xud[:, H:]


def _row_spec(d):
    return pl.BlockSpec((RB, d), lambda i: (i, 0))


def _w_spec(a):
    nd = a.ndim
    return pl.BlockSpec(a.shape, lambda i, _n=nd: (0,) * _n)


def _dense_pre(me, ms, ma, dsm, das, weights):
    f32 = jnp.float32
    in_specs = ([_row_spec(768)] + [_row_spec(256)] * 4
                + [_w_spec(w) for w in weights])
    out_shape = ([jax.ShapeDtypeStruct((NP, 128), f32)] * 2
                 + [jax.ShapeDtypeStruct((NP, H), f32)] * 4)
    out_specs = [_row_spec(128)] * 2 + [_row_spec(H)] * 4
    return pl.pallas_call(
        _pre_body,
        grid=(GRID,),
        in_specs=in_specs,
        out_specs=out_specs,
        out_shape=out_shape,
        compiler_params=pltpu.CompilerParams(
            dimension_semantics=("arbitrary",)),
    )(me, ms, ma, dsm, das, *weights)


def _sc_segsum(tab0, tab1, src_r, dst_r):
    f32 = jnp.float32
    mesh = plsc.VectorSubcoreMesh(core_axis_name="c", subcore_axis_name="s")

    @functools.partial(
        pl.kernel,
        mesh=mesh,
        compiler_params=pltpu.CompilerParams(use_tc_tiling_on_sc=False),
        out_type=[jax.ShapeDtypeStruct((2, 2, NP, H), f32),
                  jax.ShapeDtypeStruct((2, NP, H), f32)],
        scratch_types=[
            pltpu.VMEM((32, K), jnp.int32),
            pltpu.VMEM((32, K), jnp.int32),
            pltpu.VMEM((K, H), f32),
            pltpu.VMEM((K, H), f32),
            pltpu.VMEM_SHARED((NP, H), f32),
            pltpu.SemaphoreType.DMA,
            pltpu.SemaphoreType.DMA,
            pltpu.SemaphoreType.DMA,
            pltpu.SemaphoreType.DMA,
        ],
    )
    def k(tab0_h, tab1_h, src_h, dst_h, sums_o, cnts_o,
          srcv, dstv, rows0, rows1, acc, sg0, sg1, ss0, ss1):
        c = lax.axis_index("c")
        s = lax.axis_index("s")
        row0 = s * RPT
        base = (c * 16 + s) * CPT
        zero16 = jnp.zeros((16,), f32)
        one16 = jnp.ones((16,), f32)

        def fill0(val16):
            def f(i, _):
                rows0[i // 4, pl.ds((i % 4) * 16, 16)] = val16
                return 0
            lax.fori_loop(0, K * 4, f, 0)

        for p in range(3):
            fill0(zero16)
            hz = [pltpu.async_copy(rows0, acc.at[pl.ds(row0 + q * K, K)],
                                   ss0) for q in range(12)]
            hz.append(pltpu.async_copy(rows0.at[pl.ds(0, 32)],
                                       acc.at[pl.ds(row0 + 12 * K, 32)],
                                       ss0))
            for h in hz:
                h.wait()
            if p == 2:
                fill0(one16)
            plsc.subcore_barrier()

            for off, ns in ((0, 32), (32, 32), (64, 32), (96, 8)):
                if p < 2:
                    pltpu.sync_copy(src_h.at[pl.ds(base + off, ns)],
                                    srcv.at[pl.ds(0, ns)])
                pltpu.sync_copy(dst_h.at[pl.ds(base + off, ns)],
                                dstv.at[pl.ds(0, ns)])
                if p < 2:
                    tab = (tab0_h, tab1_h)[p]

                    def pair(j2, _, _tab=tab):
                        a = 2 * j2
                        b = a + 1
                        ga = pltpu.async_copy(_tab.at[srcv.at[a]], rows0,
                                              sg0)
                        gb = pltpu.async_copy(_tab.at[srcv.at[b]], rows1,
                                              sg1)
                        ga.wait()
                        sa = pltpu.async_copy(rows0, acc.at[dstv.at[a]],
                                              ss0, add=True)
                        gb.wait()
                        sb = pltpu.async_copy(rows1, acc.at[dstv.at[b]],
                                              ss1, add=True)
                        sa.wait()
                        sb.wait()
                        return 0
                    lax.fori_loop(0, ns // 2, pair, 0)
                else:
                    def pairc(j2, _):
                        a = 2 * j2
                        b = a + 1
                        sa = pltpu.async_copy(rows0, acc.at[dstv.at[a]],
                                              ss0, add=True)
                        sb = pltpu.async_copy(rows0, acc.at[dstv.at[b]],
                                              ss1, add=True)
                        sa.wait()
                        sb.wait()
                        return 0
                    lax.fori_loop(0, ns // 2, pairc, 0)
            plsc.subcore_barrier()

            if p < 2:
                pltpu.sync_copy(acc.at[pl.ds(row0, RPT)],
                                sums_o.at[c, p, pl.ds(row0, RPT)])
            else:
                pltpu.sync_copy(acc.at[pl.ds(row0, RPT)],
                                cnts_o.at[c, pl.ds(row0, RPT)])

    return k(tab0, tab1, src_r, dst_r)


def _fin_body(sd, cd, sm, cm, xomr, xodr, xum0, xum1, xud0, xud1,
              Wm2dl, bm2dl, Wm2dr, Wd2ml, bd2ml, Wd2mr, wm_raw, wd_raw,
              aggm_o, aggd_o):
    def softmax2(wr):
        a, b = wr[0, 0], wr[0, 1]
        m = jnp.maximum(a, b)
        ea, eb = jnp.exp(a - m), jnp.exp(b - m)
        return ea / (ea + eb), eb / (ea + eb)

    def leaky(x):
        return jnp.where(x >= 0, x, 0.2 * x)

    sd_, sm_ = sd[...], sm[...]
    cnt_d = jnp.maximum(cd[0, :, 0:1] + cd[1, :, 0:1], 1.0)
    cnt_m = jnp.maximum(cm[0, :, 0:1] + cm[1, :, 0:1], 1.0)
    mean_d = jnp.concatenate([sd_[0, 0] + sd_[1, 0],
                              sd_[0, 1] + sd_[1, 1]], axis=1) / cnt_d
    mean_m = jnp.concatenate([sm_[0, 0] + sm_[1, 0],
                              sm_[0, 1] + sm_[1, 1]], axis=1) / cnt_m
    xud = jnp.concatenate([xud0[...], xud1[...]], axis=1)
    xum = jnp.concatenate([xum0[...], xum1[...]], axis=1)
    out_d = leaky(_dot(mean_d, Wm2dl[...]) + bm2dl[...]
                  + _dot(xud, Wm2dr[...]))
    out_m = leaky(_dot(mean_m, Wd2ml[...]) + bd2ml[...]
                  + _dot(xum, Wd2mr[...]))
    wm0, wm1 = softmax2(wm_raw[...])
    wd0, wd1 = softmax2(wd_raw[...])
    aggm_o[...] = xomr[...] * wm0 + out_m * wm1
    aggd_o[...] = xodr[...] * wd0 + out_d * wd1


def _final(sums_d, cnts_d, sums_m, cnts_m, xomr, xodr,
           xum0, xum1, xud0, xud1, weights):
    f32 = jnp.float32
    sum_spec = pl.BlockSpec((2, 2, RB, H), lambda i: (0, 0, i, 0))
    cnt_spec = pl.BlockSpec((2, RB, H), lambda i: (0, i, 0))
    in_specs = ([sum_spec, cnt_spec, sum_spec, cnt_spec]
                + [_row_spec(128)] * 2 + [_row_spec(H)] * 4
                + [_w_spec(w) for w in weights])
    out_shape = [jax.ShapeDtypeStruct((NP, 128), f32)] * 2
    out_specs = [_row_spec(128)] * 2
    return pl.pallas_call(
        _fin_body,
        grid=(GRID,),
        in_specs=in_specs,
        out_specs=out_specs,
        out_shape=out_shape,
        compiler_params=pltpu.CompilerParams(
            dimension_semantics=("arbitrary",)),
    )(sums_d, cnts_d, sums_m, cnts_m, xomr, xodr,
      xum0, xum1, xud0, xud1, *weights)


def kernel(m_emb, m_sim, m_ass, d_sim, d_ass, edge_m2d, edge_d2m, params):
    p = params
    f32 = jnp.float32

    def padr(x):
        return jnp.pad(x, ((0, NP - N), (0, 0)))

    def b2d(b):
        return b.reshape(1, -1)

    scal = jnp.concatenate([
        p['w_emb1'], p['w_sim1'], p['w_ass1'], p['w_dsim1'], p['w_dass1'],
        p['w_emb2'], p['w_sim2'], p['w_ass2'], p['w_dsim2'], p['w_dass2'],
        jnp.zeros((6,), f32)]).reshape(1, 16)

    pre_w = (p['upd_emb_W'], b2d(p['upd_emb_b']),
             p['upd_msim_W'], b2d(p['upd_msim_b']),
             p['upd_mass_W'], b2d(p['upd_mass_b']),
             p['upd_dsim_W'], b2d(p['upd_dsim_b']),
             p['upd_dass_W'], b2d(p['upd_dass_b']),
             p['lin_emb_W'], b2d(p['lin_emb_b']),
             p['lin_msim_W'], b2d(p['lin_msim_b']),
             p['lin_mass_W'], b2d(p['lin_mass_b']),
             p['lin_dsim_W'], b2d(p['lin_dsim_b']),
             p['lin_dass_W'], b2d(p['lin_dass_b']),
             p['lin_emb_upd_W'], b2d(p['lin_emb_upd_b']),
             p['reshape_W'], b2d(p['reshape_b']),
             scal)

    xomr, xodr, xum0, xum1, xud0, xud1 = _dense_pre(
        padr(m_emb), padr(m_sim), padr(m_ass), padr(d_sim), padr(d_ass),
        pre_w)

    def prep_edges(edge):
        src = jnp.concatenate(
            [edge[0], jnp.zeros((EP - E,), jnp.int32)]).reshape(EP // K, K)
        dst = jnp.concatenate(
            [edge[1], jnp.full((EP - E,), TRASH, jnp.int32)]).reshape(
                EP // K, K)
        return src, dst

    src_m2d, dst_m2d = prep_edges(edge_m2d)
    src_d2m, dst_d2m = prep_edges(edge_d2m)

    sums_d, cnts_d = _sc_segsum(xum0, xum1, src_m2d, dst_m2d)
    sums_m, cnts_m = _sc_segsum(xud0, xud1, src_d2m, dst_d2m)

    fin_w = (p['m2d_l_W'], b2d(p['m2d_l_b']), p['m2d_r_W'],
             p['d2m_l_W'], b2d(p['d2m_l_b']), p['d2m_r_W'],
             p['weights_m'].reshape(1, 2), p['weights_d'].reshape(1, 2))

    aggm, aggd = _final(sums_d, cnts_d, sums_m, cnts_m, xomr, xodr,
                        xum0, xum1, xud0, xud1, fin_w)
    return aggm[:N], aggd[:N]


# speedup vs baseline: 1.6700x; 1.6700x over previous
import functools

import jax
import jax.numpy as jnp
from jax import lax
from jax.experimental import pallas as pl
from jax.experimental.pallas import tpu as pltpu
from jax.experimental.pallas import tpu_sc as plsc

N = 25000
NP = 25088
E = 400000
EP = 425984
CH = 64
CPT = 208
RPT = NP // 16
TRASH = N
RB = 512
GRID = NP // RB
H = 64


def _dot(a, b):
    return jnp.dot(a, b, preferred_element_type=jnp.float32)


def _pre_body(me, ms, ma, dsm, das,
              Wue, bue, Wums, bums, Wuma, buma, Wuds, buds, Wuda, buda,
              Wle, ble, Wlms, blms, Wlma, blma, Wlds, blds, Wlda, blda,
              Wleu, bleu, Wr, br, scal,
              xomr_o, xodr_o, xum0_o, xum1_o, xud0_o, xud1_o,
              xmbf_o, xdbf_o):
    s = scal[...]
    me_, ms_, ma_, dsm_, das_ = me[...], ms[...], ma[...], dsm[...], das[...]
    emb_p = _dot(me_, Wle[...]) + ble[...]
    sim_p = _dot(ms_, Wlms[...]) + blms[...]
    ass_p = _dot(ma_, Wlma[...]) + blma[...]
    dsim_p = _dot(dsm_, Wlds[...]) + blds[...]
    dass_p = _dot(das_, Wlda[...]) + blda[...]
    xom = emb_p * s[0, 0] + sim_p * s[0, 1] + ass_p * s[0, 2]
    xod = dsim_p * s[0, 3] + dass_p * s[0, 4]
    xomr_o[...] = _dot(xom, Wr[...]) + br[...]
    xodr_o[...] = _dot(xod, Wr[...]) + br[...]
    ue = jnp.tanh(_dot(me_, Wue[...]) + bue[...])
    ums = jnp.tanh(_dot(ms_, Wums[...]) + bums[...])
    uma = jnp.tanh(_dot(ma_, Wuma[...]) + buma[...])
    uds = jnp.tanh(_dot(dsm_, Wuds[...]) + buds[...])
    uda = jnp.tanh(_dot(das_, Wuda[...]) + buda[...])
    emb_u = _dot(ue, Wleu[...]) + bleu[...]
    sim_u = _dot(ums, Wlms[...]) + blms[...]
    ass_u = _dot(uma, Wlma[...]) + blma[...]
    dsim_u = _dot(uds, Wlds[...]) + blds[...]
    dass_u = _dot(uda, Wlda[...]) + blda[...]
    xum = emb_u * s[0, 5] + sim_u * s[0, 6] + ass_u * s[0, 7]
    xud = dsim_u * s[0, 8] + dass_u * s[0, 9]
    xum0_o[...] = xum[:, :H]
    xum1_o[...] = xum[:, H:]
    xud0_o[...] = xud[:, :H]
    xud1_o[...] = xud[:, H:]
    xmbf_o[...] = xum.astype(jnp.bfloat16)
    xdbf_o[...] = xud.astype(jnp.bfloat16)


def _row_spec(d):
    return pl.BlockSpec((RB, d), lambda i: (i, 0))


def _w_spec(a):
    nd = a.ndim
    return pl.BlockSpec(a.shape, lambda i, _n=nd: (0,) * _n)


def _dense_pre(me, ms, ma, dsm, das, weights):
    f32 = jnp.float32
    in_specs = ([_row_spec(768)] + [_row_spec(256)] * 4
                + [_w_spec(w) for w in weights])
    out_shape = ([jax.ShapeDtypeStruct((NP, 128), f32)] * 2
                 + [jax.ShapeDtypeStruct((NP, H), f32)] * 4
                 + [jax.ShapeDtypeStruct((NP, 128), jnp.bfloat16)] * 2)
    out_specs = [_row_spec(128)] * 2 + [_row_spec(H)] * 4 + \
        [_row_spec(128)] * 2
    return pl.pallas_call(
        _pre_body,
        grid=(GRID,),
        in_specs=in_specs,
        out_specs=out_specs,
        out_shape=out_shape,
        compiler_params=pltpu.CompilerParams(
            dimension_semantics=("arbitrary",)),
    )(me, ms, ma, dsm, das, *weights)


def _sc_segsum(tab, src_r, dst_r):
    bf16 = jnp.bfloat16
    mesh = plsc.VectorSubcoreMesh(core_axis_name="c", subcore_axis_name="s")

    @functools.partial(
        pl.kernel,
        mesh=mesh,
        compiler_params=pltpu.CompilerParams(use_tc_tiling_on_sc=False),
        out_type=[jax.ShapeDtypeStruct((2, NP, 2 * H), bf16),
                  jax.ShapeDtypeStruct((2, NP, 2 * H), bf16)],
        scratch_types=[
            pltpu.VMEM((32, CH), jnp.int32),
            pltpu.VMEM((32, CH), jnp.int32),
            pltpu.VMEM((CH, 2 * H), bf16),
            pltpu.VMEM((CH, 2 * H), bf16),
            pltpu.VMEM((CH, 2 * H), bf16),
            pltpu.VMEM((CH, 2 * H), bf16),
            pltpu.VMEM_SHARED((NP, 2 * H), bf16),
            pltpu.SemaphoreType.DMA,
            pltpu.SemaphoreType.DMA,
            pltpu.SemaphoreType.DMA,
            pltpu.SemaphoreType.DMA,
            pltpu.SemaphoreType.DMA,
            pltpu.SemaphoreType.DMA,
            pltpu.SemaphoreType.DMA,
            pltpu.SemaphoreType.DMA,
        ],
    )
    def k(tab_h, src_h, dst_h, sums_o, cnts_o,
          srcv, dstv, r0, r1, r2, r3,
          acc, sg0, sg1, sg2, sg3, ss0, ss1, ss2, ss3):
        c = lax.axis_index("c")
        s = lax.axis_index("s")
        row0 = s * RPT
        base = (c * 16 + s) * CPT
        rows = (r0, r1, r2, r3)
        sgs = (sg0, sg1, sg2, sg3)
        sss = (ss0, ss1, ss2, ss3)
        zero32 = jnp.zeros((32,), bf16)
        one32 = jnp.ones((32,), bf16)

        def fill(buf, val32):
            def f(i, _):
                buf[i // 4, pl.ds((i % 4) * 32, 32)] = val32
                return 0
            lax.fori_loop(0, CH * 4, f, 0)

        for p in range(2):
            fill(r0, zero32)
            hz = [pltpu.async_copy(r0, acc.at[pl.ds(row0 + q * CH, CH)],
                                   ss0) for q in range(24)]
            hz.append(pltpu.async_copy(r0.at[pl.ds(0, 32)],
                                       acc.at[pl.ds(row0 + 24 * CH, 32)],
                                       ss0))
            for h in hz:
                h.wait()
            if p == 1:
                fill(r0, one32)
            plsc.subcore_barrier()

            for off, ns in ((0, 32), (32, 32), (64, 32), (96, 32),
                            (128, 32), (160, 32), (192, 16)):
                if p == 0:
                    pltpu.sync_copy(src_h.at[pl.ds(base + off, ns)],
                                    srcv.at[pl.ds(0, ns)])
                pltpu.sync_copy(dst_h.at[pl.ds(base + off, ns)],
                                dstv.at[pl.ds(0, ns)])
                if p == 0:
                    def quad(j4, _):
                        a = 4 * j4
                        gs = [pltpu.async_copy(tab_h.at[srcv.at[a + q]],
                                               rows[q], sgs[q])
                              for q in range(4)]
                        scs = []
                        for q in range(4):
                            gs[q].wait()
                            scs.append(pltpu.async_copy(
                                rows[q], acc.at[dstv.at[a + q]], sss[q],
                                add=True))
                        for h in scs:
                            h.wait()
                        return 0
                    lax.fori_loop(0, ns // 4, quad, 0)
                else:
                    def quadc(j4, _):
                        a = 4 * j4
                        scs = [pltpu.async_copy(
                            r0, acc.at[dstv.at[a + q]], sss[q], add=True)
                            for q in range(4)]
                        for h in scs:
                            h.wait()
                        return 0
                    lax.fori_loop(0, ns // 4, quadc, 0)
            plsc.subcore_barrier()

            out = sums_o if p == 0 else cnts_o
            pltpu.sync_copy(acc.at[pl.ds(row0, RPT)],
                            out.at[c, pl.ds(row0, RPT)])

    return k(tab, src_r, dst_r)


def _fin_body(sd, cd, sm, cm, xomr, xodr, xum0, xum1, xud0, xud1,
              Wm2dl, bm2dl, Wm2dr, Wd2ml, bd2ml, Wd2mr, wm_raw, wd_raw,
              aggm_o, aggd_o):
    def softmax2(wr):
        a, b = wr[0, 0], wr[0, 1]
        m = jnp.maximum(a, b)
        ea, eb = jnp.exp(a - m), jnp.exp(b - m)
        return ea / (ea + eb), eb / (ea + eb)

    def leaky(x):
        return jnp.where(x >= 0, x, 0.2 * x)

    f32 = jnp.float32
    sd_, sm_ = sd[...].astype(f32), sm[...].astype(f32)
    cd_, cm_ = cd[...].astype(f32), cm[...].astype(f32)
    cnt_d = jnp.maximum(cd_[0, :, 0:1] + cd_[1, :, 0:1], 1.0)
    cnt_m = jnp.maximum(cm_[0, :, 0:1] + cm_[1, :, 0:1], 1.0)
    mean_d = (sd_[0] + sd_[1]) / cnt_d
    mean_m = (sm_[0] + sm_[1]) / cnt_m
    xud = jnp.concatenate([xud0[...], xud1[...]], axis=1)
    xum = jnp.concatenate([xum0[...], xum1[...]], axis=1)
    out_d = leaky(_dot(mean_d, Wm2dl[...]) + bm2dl[...]
                  + _dot(xud, Wm2dr[...]))
    out_m = leaky(_dot(mean_m, Wd2ml[...]) + bd2ml[...]
                  + _dot(xum, Wd2mr[...]))
    wm0, wm1 = softmax2(wm_raw[...])
    wd0, wd1 = softmax2(wd_raw[...])
    aggm_o[...] = xomr[...] * wm0 + out_m * wm1
    aggd_o[...] = xodr[...] * wd0 + out_d * wd1


def _final(sums_d, cnts_d, sums_m, cnts_m, xomr, xodr,
           xum0, xum1, xud0, xud1, weights):
    f32 = jnp.float32
    sum_spec = pl.BlockSpec((2, RB, 2 * H), lambda i: (0, i, 0))
    cnt_spec = pl.BlockSpec((2, RB, 2 * H), lambda i: (0, i, 0))
    in_specs = ([sum_spec, cnt_spec, sum_spec, cnt_spec]
                + [_row_spec(128)] * 2 + [_row_spec(H)] * 4
                + [_w_spec(w) for w in weights])
    out_shape = [jax.ShapeDtypeStruct((NP, 128), f32)] * 2
    out_specs = [_row_spec(128)] * 2
    return pl.pallas_call(
        _fin_body,
        grid=(GRID,),
        in_specs=in_specs,
        out_specs=out_specs,
        out_shape=out_shape,
        compiler_params=pltpu.CompilerParams(
            dimension_semantics=("arbitrary",)),
    )(sums_d, cnts_d, sums_m, cnts_m, xomr, xodr,
      xum0, xum1, xud0, xud1, *weights)


def kernel(m_emb, m_sim, m_ass, d_sim, d_ass, edge_m2d, edge_d2m, params):
    p = params
    f32 = jnp.float32

    def padr(x):
        return jnp.pad(x, ((0, NP - N), (0, 0)))

    def b2d(b):
        return b.reshape(1, -1)

    scal = jnp.concatenate([
        p['w_emb1'], p['w_sim1'], p['w_ass1'], p['w_dsim1'], p['w_dass1'],
        p['w_emb2'], p['w_sim2'], p['w_ass2'], p['w_dsim2'], p['w_dass2'],
        jnp.zeros((6,), f32)]).reshape(1, 16)

    pre_w = (p['upd_emb_W'], b2d(p['upd_emb_b']),
             p['upd_msim_W'], b2d(p['upd_msim_b']),
             p['upd_mass_W'], b2d(p['upd_mass_b']),
             p['upd_dsim_W'], b2d(p['upd_dsim_b']),
             p['upd_dass_W'], b2d(p['upd_dass_b']),
             p['lin_emb_W'], b2d(p['lin_emb_b']),
             p['lin_msim_W'], b2d(p['lin_msim_b']),
             p['lin_mass_W'], b2d(p['lin_mass_b']),
             p['lin_dsim_W'], b2d(p['lin_dsim_b']),
             p['lin_dass_W'], b2d(p['lin_dass_b']),
             p['lin_emb_upd_W'], b2d(p['lin_emb_upd_b']),
             p['reshape_W'], b2d(p['reshape_b']),
             scal)

    xomr, xodr, xum0, xum1, xud0, xud1, xmbf, xdbf = _dense_pre(
        padr(m_emb), padr(m_sim), padr(m_ass), padr(d_sim), padr(d_ass),
        pre_w)

    def prep_edges(edge):
        src = jnp.concatenate(
            [edge[0], jnp.zeros((EP - E,), jnp.int32)]).reshape(EP // CH, CH)
        dst = jnp.concatenate(
            [edge[1], jnp.full((EP - E,), TRASH, jnp.int32)]).reshape(
                EP // CH, CH)
        return src, dst

    src_m2d, dst_m2d = prep_edges(edge_m2d)
    src_d2m, dst_d2m = prep_edges(edge_d2m)

    sums_d, cnts_d = _sc_segsum(xmbf, src_m2d, dst_m2d)
    sums_m, cnts_m = _sc_segsum(xdbf, src_d2m, dst_d2m)

    fin_w = (p['m2d_l_W'], b2d(p['m2d_l_b']), p['m2d_r_W'],
             p['d2m_l_W'], b2d(p['d2m_l_b']), p['d2m_r_W'],
             p['weights_m'].reshape(1, 2), p['weights_d'].reshape(1, 2))

    aggm, aggd = _final(sums_d, cnts_d, sums_m, cnts_m, xomr, xodr,
                        xum0, xum1, xud0, xud1, fin_w)
    return aggm[:N], aggd[:N]
